# Initial kernel scaffold; baseline (speedup 1.0000x reference)
import jax
import jax.numpy as jnp
from jax.experimental import pallas as pl


def kernel(x, edge_index, W1_l, W1_r, b1, W2_l, W2_r, b2):
    raise NotImplementedError("write your pallas kernel here")


# speedup vs baseline: 6.7110x; 6.7110x over previous
import functools
import jax
import jax.numpy as jnp
from jax import lax
from jax.experimental import pallas as pl
from jax.experimental.pallas import tpu as pltpu
from jax.experimental.pallas import tpu_sc as plsc

N = 10000
E = 160000
D = 256
H = 256

NC = 2
NS = 16
CHUNK = 128
EP = 163840
CPT = EP // NS // CHUNK
NPAD = 10240
RPT = NPAD // NS


def _sc_mesh():
    return plsc.VectorSubcoreMesh(core_axis_name="c", subcore_axis_name="s",
                                  num_cores=NC, num_subcores=NS)


P = 64


def _aggr_body(x00_hbm, x01_hbm, x10_hbm, x11_hbm, src_hbm, dst_hbm,
               zrow_hbm, iota_hbm,
               a0_hbm, a1_hbm, a2_hbm, a3_hbm, deg_hbm,
               aggr_sh, deg_sh, src_v, dst_v, rows_v, deg_l, rep_v, iota_v):
    cid = lax.axis_index("c")
    sid = lax.axis_index("s")

    pltpu.sync_copy(src_hbm.at[sid], src_v)
    pltpu.sync_copy(dst_hbm.at[sid], dst_v)
    pltpu.sync_copy(iota_hbm, iota_v)

    zero16 = jnp.zeros((16,), jnp.float32)

    def zdeg(i, _):
        deg_l[pl.ds(i * 16, 16)] = zero16
        return 0
    lax.fori_loop(0, RPT, zdeg, 0)

    def zrep(i, _):
        rep_v[i] = zero16
        return 0
    lax.fori_loop(0, CHUNK, zrep, 0)

    @pl.when(jnp.logical_and(cid == 0, sid == 0))
    def _():
        for j in range(RPT // CHUNK):
            pltpu.sync_copy(rep_v, deg_sh.at[pl.ds(j * CHUNK, CHUNK)])

    sl = pl.ds(sid * RPT, RPT)

    def run_pass(x_hbm, out_hbm):
        pltpu.sync_copy(zrow_hbm, aggr_sh.at[sl])
        plsc.subcore_barrier()

        def body(c, _):
            pltpu.sync_copy(x_hbm.at[src_v.at[c]], rows_v)
            pltpu.sync_copy(rows_v, aggr_sh.at[dst_v.at[c]], add=True)
            return 0
        lax.fori_loop(0, CPT, body, 0)
        plsc.subcore_barrier()
        pltpu.sync_copy(aggr_sh.at[sl], out_hbm.at[sl])

    @pl.when(cid == 0)
    def _():
        run_pass(x00_hbm, a0_hbm)
        run_pass(x01_hbm, a1_hbm)

    @pl.when(cid == 1)
    def _():
        run_pass(x10_hbm, a2_hbm)
        run_pass(x11_hbm, a3_hbm)

    @pl.when(cid == 0)
    def _():
        ones16 = jnp.ones((16,), jnp.float32)

        def dbody(c, _):
            for j in range(CHUNK // 16):
                d16 = dst_v[c, pl.ds(j * 16, 16)]
                plsc.addupdate_scatter(deg_l, [d16], ones16)
            return 0
        lax.fori_loop(0, CPT, dbody, 0)
        for j in range(RPT // CHUNK):
            def rbody(i, _):
                rep_v[i] = deg_l[pl.ds(j * CHUNK * 16 + i * 16, 16)]
                return 0
            lax.fori_loop(0, CHUNK, rbody, 0)
            pltpu.sync_copy(rep_v, deg_sh.at[iota_v.at[j]], add=True)
        plsc.subcore_barrier()
        pltpu.sync_copy(deg_sh.at[pl.ds(sid * (RPT // 16), RPT // 16)],
                        deg_hbm.at[pl.ds(sid * (RPT // 16), RPT // 16)])


def _make_aggr_kernel():
    return pl.kernel(
        _aggr_body,
        out_type=(
            jax.ShapeDtypeStruct((NPAD, P), jnp.float32),
            jax.ShapeDtypeStruct((NPAD, P), jnp.float32),
            jax.ShapeDtypeStruct((NPAD, P), jnp.float32),
            jax.ShapeDtypeStruct((NPAD, P), jnp.float32),
            jax.ShapeDtypeStruct((NPAD // 16, 16), jnp.float32),
        ),
        mesh=_sc_mesh(),
        scratch_types=[
            pltpu.VMEM_SHARED((NPAD, P), jnp.float32),
            pltpu.VMEM_SHARED((NPAD // 16, 16), jnp.float32),
            pltpu.VMEM((CPT, CHUNK), jnp.int32),
            pltpu.VMEM((CPT, CHUNK), jnp.int32),
            pltpu.VMEM((CHUNK, P), jnp.float32),
            pltpu.VMEM((NPAD,), jnp.float32),
            pltpu.VMEM((CHUNK, 16), jnp.float32),
            pltpu.VMEM((RPT // CHUNK, CHUNK), jnp.int32),
        ],
        compiler_params=pltpu.CompilerParams(needs_layout_passes=False, use_tc_tiling_on_sc=False),
    )


def _dense_body(a0_ref, a1_ref, a2_ref, a3_ref, x_ref, deg_ref,
                w1l0_ref, w1l1_ref, w1l2_ref, w1l3_ref,
                w1r_ref, b1_ref, w2_ref, b2_ref, s_ref):
    inv = 1.0 / jnp.maximum(deg_ref[...], 1.0)
    f32 = jnp.float32
    h = (jnp.dot(a0_ref[...] * inv, w1l0_ref[...], preferred_element_type=f32)
         + jnp.dot(a1_ref[...] * inv, w1l1_ref[...], preferred_element_type=f32)
         + jnp.dot(a2_ref[...] * inv, w1l2_ref[...], preferred_element_type=f32)
         + jnp.dot(a3_ref[...] * inv, w1l3_ref[...], preferred_element_type=f32)
         + jnp.dot(x_ref[...], w1r_ref[...], preferred_element_type=f32)
         + b1_ref[...])
    h = jnp.maximum(h, 0.0)
    s_ref[...] = (jnp.dot(h, w2_ref[...], preferred_element_type=f32)
                  + b2_ref[...])


def _make_dense_kernel():
    nb = NPAD // 256
    return pl.pallas_call(
        _dense_body,
        grid=(nb,),
        in_specs=[
            pl.BlockSpec((256, P), lambda i: (i, 0)),
            pl.BlockSpec((256, P), lambda i: (i, 0)),
            pl.BlockSpec((256, P), lambda i: (i, 0)),
            pl.BlockSpec((256, P), lambda i: (i, 0)),
            pl.BlockSpec((256, D), lambda i: (i, 0)),
            pl.BlockSpec((256, 1), lambda i: (i, 0)),
            pl.BlockSpec((P, H), lambda i: (0, 0)),
            pl.BlockSpec((P, H), lambda i: (0, 0)),
            pl.BlockSpec((P, H), lambda i: (0, 0)),
            pl.BlockSpec((P, H), lambda i: (0, 0)),
            pl.BlockSpec((D, H), lambda i: (0, 0)),
            pl.BlockSpec((1, H), lambda i: (0, 0)),
            pl.BlockSpec((H, 2), lambda i: (0, 0)),
            pl.BlockSpec((1, 2), lambda i: (0, 0)),
        ],
        out_specs=pl.BlockSpec((256, 2), lambda i: (i, 0)),
        out_shape=jax.ShapeDtypeStruct((NPAD, 2), jnp.float32),
    )


def _seg2_body(s_hbm, src_hbm, dst_hbm, deg_hbm, iota_hbm,
               out_hbm,
               t_sh, s_v, src_v, dst_v, deg_v, t_l, rep_v, tb_v, out_v,
               iota_v):
    cid = lax.axis_index("c")
    sid = lax.axis_index("s")

    pltpu.sync_copy(s_hbm, s_v)
    pltpu.sync_copy(src_hbm.at[sid], src_v)
    pltpu.sync_copy(dst_hbm.at[sid], dst_v)
    pltpu.sync_copy(deg_hbm, deg_v)
    pltpu.sync_copy(iota_hbm, iota_v)

    zero16 = jnp.zeros((16,), jnp.float32)

    def zt(i, _):
        t_l[pl.ds(i * 16, 16)] = zero16
        return 0
    lax.fori_loop(0, RPT, zt, 0)

    def zrep(i, _):
        rep_v[i] = zero16
        return 0
    lax.fori_loop(0, CHUNK, zrep, 0)

    @pl.when(sid == 0)
    def _():
        for j in range(RPT // CHUNK):
            pltpu.sync_copy(rep_v, t_sh.at[pl.ds(j * CHUNK, CHUNK)])

    plsc.subcore_barrier()

    def body(c, _):
        for j in range(CHUNK // 16):
            s16 = src_v[c, pl.ds(j * 16, 16)]
            d16 = dst_v[c, pl.ds(j * 16, 16)]
            v = plsc.load_gather(s_v, [lax.shift_left(s16, 1)])
            plsc.addupdate_scatter(t_l, [d16], v)
        return 0
    lax.fori_loop(0, CPT, body, 0)

    for j in range(RPT // CHUNK):
        def rbody(i, _):
            rep_v[i] = t_l[pl.ds(j * CHUNK * 16 + i * 16, 16)]
            return 0
        lax.fori_loop(0, CHUNK, rbody, 0)
        pltpu.sync_copy(rep_v, t_sh.at[iota_v.at[j]], add=True)

    plsc.subcore_barrier()

    @pl.when(cid == 0)
    def _():
        pltpu.sync_copy(t_sh.at[pl.ds(sid * (RPT // 16), RPT // 16)], tb_v)
        base = sid * RPT
        i16 = lax.iota(jnp.int32, 16)

        def fbody(j, _):
            tt = tb_v[j]
            dd = deg_v[pl.ds(base + j * 16, 16)]
            s1 = plsc.load_gather(
                s_v, [lax.shift_left(i16 + (base + j * 16), 1) + 1])
            out_v[pl.ds(j * 16, 16)] = tt / jnp.maximum(dd, 1.0) + s1
            return 0
        lax.fori_loop(0, RPT // 16, fbody, 0)
        pltpu.sync_copy(out_v, out_hbm.at[pl.ds(base, RPT)])


def _make_seg2_kernel():
    return pl.kernel(
        _seg2_body,
        out_type=jax.ShapeDtypeStruct((NPAD,), jnp.float32),
        mesh=_sc_mesh(),
        scratch_types=[
            pltpu.VMEM_SHARED((NPAD // 16, 16), jnp.float32),
            pltpu.VMEM((NPAD * 2,), jnp.float32),
            pltpu.VMEM((CPT, CHUNK), jnp.int32),
            pltpu.VMEM((CPT, CHUNK), jnp.int32),
            pltpu.VMEM((NPAD,), jnp.float32),
            pltpu.VMEM((NPAD,), jnp.float32),
            pltpu.VMEM((CHUNK, 16), jnp.float32),
            pltpu.VMEM((RPT // 16, 16), jnp.float32),
            pltpu.VMEM((RPT,), jnp.float32),
            pltpu.VMEM((RPT // CHUNK, CHUNK), jnp.int32),
        ],
        compiler_params=pltpu.CompilerParams(needs_layout_passes=False, use_tc_tiling_on_sc=False),
    )


@jax.jit
def kernel(x, edge_index, W1_l, W1_r, b1, W2_l, W2_r, b2):
    npad = EP - E
    pad_src = (jnp.arange(npad, dtype=jnp.int32) * 37) % N
    pad_dst = N + (jnp.arange(npad, dtype=jnp.int32) % (NPAD - N))
    src = jnp.concatenate([edge_index[0].astype(jnp.int32), pad_src])
    dst = jnp.concatenate([edge_index[1].astype(jnp.int32), pad_dst])
    src3 = src.reshape(NS, CPT, CHUNK)
    dst3 = dst.reshape(NS, CPT, CHUNK)

    xs = [x[:, i * P:(i + 1) * P] for i in range(4)]
    zrow = jnp.zeros((RPT, P), jnp.float32)
    iota = jnp.arange(NPAD // 16, dtype=jnp.int32).reshape(RPT // CHUNK, CHUNK)

    a0, a1, a2, a3, deg2 = _make_aggr_kernel()(
        xs[0], xs[1], xs[2], xs[3], src3, dst3, zrow, iota)
    deg = deg2.reshape(NPAD, 1)

    xp = jnp.zeros((NPAD, D), x.dtype).at[:N].set(x)
    w2 = jnp.concatenate([W2_l, W2_r], axis=0).T
    b2v = jnp.stack([jnp.zeros((), jnp.float32), b2[0]]).reshape(1, 2)
    w1lT = W1_l.T
    s = _make_dense_kernel()(
        a0, a1, a2, a3, xp, deg,
        w1lT[0 * P:1 * P], w1lT[1 * P:2 * P], w1lT[2 * P:3 * P],
        w1lT[3 * P:4 * P], W1_r.T, b1.reshape(1, H), w2, b2v)

    outf = _make_seg2_kernel()(s.reshape(NPAD * 2), src3, dst3,
                               deg.reshape(NPAD), iota)
    return outf[:N].reshape(N, 1)


# speedup vs baseline: 9.0450x; 1.3478x over previous
import functools
import jax
import jax.numpy as jnp
from jax import lax
from jax.experimental import pallas as pl
from jax.experimental.pallas import tpu as pltpu
from jax.experimental.pallas import tpu_sc as plsc

N = 10000
E = 160000
D = 256
H = 256

NC = 2
NS = 16
CHUNK = 128
EP = 163840
CPT = EP // NS // CHUNK
NPAD = 10240
RPT = NPAD // NS


def _sc_mesh():
    return plsc.VectorSubcoreMesh(core_axis_name="c", subcore_axis_name="s",
                                  num_cores=NC, num_subcores=NS)


P = 64


def _aggr_body(x00_hbm, x01_hbm, x10_hbm, x11_hbm, src_hbm, dst_hbm,
               zrow_hbm, iota_hbm,
               a0_hbm, a1_hbm, a2_hbm, a3_hbm, deg_hbm,
               aggr_sh, deg_sh, src_v, dst_v, rows_a, rows_b, deg_l, rep_v,
               iota_v, sem_a, sem_b):
    cid = lax.axis_index("c")
    sid = lax.axis_index("s")

    pltpu.sync_copy(src_hbm.at[sid], src_v)
    pltpu.sync_copy(dst_hbm.at[sid], dst_v)
    pltpu.sync_copy(iota_hbm, iota_v)

    zero16 = jnp.zeros((16,), jnp.float32)

    def zdeg(i, _):
        deg_l[pl.ds(i * 16, 16)] = zero16
        return 0
    lax.fori_loop(0, RPT, zdeg, 0)

    def zrep(i, _):
        rep_v[i] = zero16
        return 0
    lax.fori_loop(0, CHUNK, zrep, 0)

    @pl.when(jnp.logical_and(cid == 0, sid == 0))
    def _():
        for j in range(RPT // CHUNK):
            pltpu.sync_copy(rep_v, deg_sh.at[pl.ds(j * CHUNK, CHUNK)])

    sl = pl.ds(sid * RPT, RPT)
    ones16 = jnp.ones((16,), jnp.float32)

    def hist(c):
        for j in range(CHUNK // 16):
            d16 = dst_v[c, pl.ds(j * 16, 16)]
            plsc.addupdate_scatter(deg_l, [d16], ones16)

    def gather(c, buf, sem):
        return pltpu.make_async_copy(x_hbm_cur[0].at[src_v.at[c]], buf, sem)

    x_hbm_cur = [None]

    def run_pass(x_hbm, out_hbm, with_deg):
        x_hbm_cur[0] = x_hbm
        pltpu.sync_copy(zrow_hbm, aggr_sh.at[sl])
        plsc.subcore_barrier()
        gather(0, rows_a, sem_a).start()

        def body(o, _):
            c0 = 2 * o
            gather(c0 + 1, rows_b, sem_b).start()
            if with_deg:
                hist(c0)
            gather(c0, rows_a, sem_a).wait()
            pltpu.sync_copy(rows_a, aggr_sh.at[dst_v.at[c0]], add=True)

            @pl.when(c0 + 2 < CPT)
            def _():
                gather(c0 + 2, rows_a, sem_a).start()
            if with_deg:
                hist(c0 + 1)
            gather(c0 + 1, rows_b, sem_b).wait()
            pltpu.sync_copy(rows_b, aggr_sh.at[dst_v.at[c0 + 1]], add=True)
            return 0
        lax.fori_loop(0, CPT // 2, body, 0)
        plsc.subcore_barrier()
        pltpu.sync_copy(aggr_sh.at[sl], out_hbm.at[sl])

    @pl.when(cid == 0)
    def _():
        run_pass(x00_hbm, a0_hbm, True)
        run_pass(x01_hbm, a1_hbm, False)

    @pl.when(cid == 1)
    def _():
        run_pass(x10_hbm, a2_hbm, False)
        run_pass(x11_hbm, a3_hbm, False)

    @pl.when(cid == 0)
    def _():
        for j in range(RPT // CHUNK):
            def rbody(i, _):
                rep_v[i] = deg_l[pl.ds(j * CHUNK * 16 + i * 16, 16)]
                return 0
            lax.fori_loop(0, CHUNK, rbody, 0)
            pltpu.sync_copy(rep_v, deg_sh.at[iota_v.at[j]], add=True)
        plsc.subcore_barrier()
        pltpu.sync_copy(deg_sh.at[pl.ds(sid * (RPT // 16), RPT // 16)],
                        deg_hbm.at[pl.ds(sid * (RPT // 16), RPT // 16)])


def _make_aggr_kernel():
    return pl.kernel(
        _aggr_body,
        out_type=(
            jax.ShapeDtypeStruct((NPAD, P), jnp.float32),
            jax.ShapeDtypeStruct((NPAD, P), jnp.float32),
            jax.ShapeDtypeStruct((NPAD, P), jnp.float32),
            jax.ShapeDtypeStruct((NPAD, P), jnp.float32),
            jax.ShapeDtypeStruct((NPAD // 16, 16), jnp.float32),
        ),
        mesh=_sc_mesh(),
        scratch_types=[
            pltpu.VMEM_SHARED((NPAD, P), jnp.float32),
            pltpu.VMEM_SHARED((NPAD // 16, 16), jnp.float32),
            pltpu.VMEM((CPT, CHUNK), jnp.int32),
            pltpu.VMEM((CPT, CHUNK), jnp.int32),
            pltpu.VMEM((CHUNK, P), jnp.float32),
            pltpu.VMEM((CHUNK, P), jnp.float32),
            pltpu.VMEM((NPAD,), jnp.float32),
            pltpu.VMEM((CHUNK, 16), jnp.float32),
            pltpu.VMEM((RPT // CHUNK, CHUNK), jnp.int32),
            pltpu.SemaphoreType.DMA,
            pltpu.SemaphoreType.DMA,
        ],
        compiler_params=pltpu.CompilerParams(needs_layout_passes=False, use_tc_tiling_on_sc=False),
    )


def _dense_body(a0_ref, a1_ref, a2_ref, a3_ref, x_ref, deg_ref,
                w1l0_ref, w1l1_ref, w1l2_ref, w1l3_ref,
                w1r_ref, b1_ref, w2_ref, b2_ref, s_ref):
    inv = 1.0 / jnp.maximum(deg_ref[...], 1.0)
    f32 = jnp.float32
    h = (jnp.dot(a0_ref[...] * inv, w1l0_ref[...], preferred_element_type=f32)
         + jnp.dot(a1_ref[...] * inv, w1l1_ref[...], preferred_element_type=f32)
         + jnp.dot(a2_ref[...] * inv, w1l2_ref[...], preferred_element_type=f32)
         + jnp.dot(a3_ref[...] * inv, w1l3_ref[...], preferred_element_type=f32)
         + jnp.dot(x_ref[...], w1r_ref[...], preferred_element_type=f32)
         + b1_ref[...])
    h = jnp.maximum(h, 0.0)
    s_ref[...] = (jnp.dot(h, w2_ref[...], preferred_element_type=f32)
                  + b2_ref[...])


def _make_dense_kernel():
    nb = NPAD // 256
    return pl.pallas_call(
        _dense_body,
        grid=(nb,),
        in_specs=[
            pl.BlockSpec((256, P), lambda i: (i, 0)),
            pl.BlockSpec((256, P), lambda i: (i, 0)),
            pl.BlockSpec((256, P), lambda i: (i, 0)),
            pl.BlockSpec((256, P), lambda i: (i, 0)),
            pl.BlockSpec((256, D), lambda i: (i, 0)),
            pl.BlockSpec((256, 1), lambda i: (i, 0)),
            pl.BlockSpec((P, H), lambda i: (0, 0)),
            pl.BlockSpec((P, H), lambda i: (0, 0)),
            pl.BlockSpec((P, H), lambda i: (0, 0)),
            pl.BlockSpec((P, H), lambda i: (0, 0)),
            pl.BlockSpec((D, H), lambda i: (0, 0)),
            pl.BlockSpec((1, H), lambda i: (0, 0)),
            pl.BlockSpec((H, 2), lambda i: (0, 0)),
            pl.BlockSpec((1, 2), lambda i: (0, 0)),
        ],
        out_specs=pl.BlockSpec((256, 2), lambda i: (i, 0)),
        out_shape=jax.ShapeDtypeStruct((NPAD, 2), jnp.float32),
    )


def _seg2_body(s_hbm, src_hbm, dst_hbm, deg_hbm, iota_hbm,
               out_hbm,
               t_sh, s_v, src_v, dst_v, deg_v, t_l, rep_v, tb_v, out_v,
               iota_v):
    cid = lax.axis_index("c")
    sid = lax.axis_index("s")

    pltpu.sync_copy(s_hbm, s_v)
    pltpu.sync_copy(src_hbm.at[sid], src_v)
    pltpu.sync_copy(dst_hbm.at[sid], dst_v)
    pltpu.sync_copy(deg_hbm, deg_v)
    pltpu.sync_copy(iota_hbm, iota_v)

    zero16 = jnp.zeros((16,), jnp.float32)

    def zt(i, _):
        t_l[pl.ds(i * 16, 16)] = zero16
        return 0
    lax.fori_loop(0, RPT, zt, 0)

    def zrep(i, _):
        rep_v[i] = zero16
        return 0
    lax.fori_loop(0, CHUNK, zrep, 0)

    @pl.when(sid == 0)
    def _():
        for j in range(RPT // CHUNK):
            pltpu.sync_copy(rep_v, t_sh.at[pl.ds(j * CHUNK, CHUNK)])

    plsc.subcore_barrier()

    def body(c, _):
        for j in range(CHUNK // 16):
            s16 = src_v[c, pl.ds(j * 16, 16)]
            d16 = dst_v[c, pl.ds(j * 16, 16)]
            v = plsc.load_gather(s_v, [lax.shift_left(s16, 1)])
            plsc.addupdate_scatter(t_l, [d16], v)
        return 0
    lax.fori_loop(0, CPT, body, 0)

    for j in range(RPT // CHUNK):
        def rbody(i, _):
            rep_v[i] = t_l[pl.ds(j * CHUNK * 16 + i * 16, 16)]
            return 0
        lax.fori_loop(0, CHUNK, rbody, 0)
        pltpu.sync_copy(rep_v, t_sh.at[iota_v.at[j]], add=True)

    plsc.subcore_barrier()

    @pl.when(cid == 0)
    def _():
        pltpu.sync_copy(t_sh.at[pl.ds(sid * (RPT // 16), RPT // 16)], tb_v)
        base = sid * RPT
        i16 = lax.iota(jnp.int32, 16)

        def fbody(j, _):
            tt = tb_v[j]
            dd = deg_v[pl.ds(base + j * 16, 16)]
            s1 = plsc.load_gather(
                s_v, [lax.shift_left(i16 + (base + j * 16), 1) + 1])
            out_v[pl.ds(j * 16, 16)] = tt / jnp.maximum(dd, 1.0) + s1
            return 0
        lax.fori_loop(0, RPT // 16, fbody, 0)
        pltpu.sync_copy(out_v, out_hbm.at[pl.ds(base, RPT)])


def _make_seg2_kernel():
    return pl.kernel(
        _seg2_body,
        out_type=jax.ShapeDtypeStruct((NPAD,), jnp.float32),
        mesh=_sc_mesh(),
        scratch_types=[
            pltpu.VMEM_SHARED((NPAD // 16, 16), jnp.float32),
            pltpu.VMEM((NPAD * 2,), jnp.float32),
            pltpu.VMEM((CPT, CHUNK), jnp.int32),
            pltpu.VMEM((CPT, CHUNK), jnp.int32),
            pltpu.VMEM((NPAD,), jnp.float32),
            pltpu.VMEM((NPAD,), jnp.float32),
            pltpu.VMEM((CHUNK, 16), jnp.float32),
            pltpu.VMEM((RPT // 16, 16), jnp.float32),
            pltpu.VMEM((RPT,), jnp.float32),
            pltpu.VMEM((RPT // CHUNK, CHUNK), jnp.int32),
        ],
        compiler_params=pltpu.CompilerParams(needs_layout_passes=False, use_tc_tiling_on_sc=False),
    )


@jax.jit
def kernel(x, edge_index, W1_l, W1_r, b1, W2_l, W2_r, b2):
    npad = EP - E
    pad_src = (jnp.arange(npad, dtype=jnp.int32) * 37) % N
    pad_dst = N + (jnp.arange(npad, dtype=jnp.int32) % (NPAD - N))
    src = jnp.concatenate([edge_index[0].astype(jnp.int32), pad_src])
    dst = jnp.concatenate([edge_index[1].astype(jnp.int32), pad_dst])
    src3 = src.reshape(NS, CPT, CHUNK)
    dst3 = dst.reshape(NS, CPT, CHUNK)

    xs = [x[:, i * P:(i + 1) * P] for i in range(4)]
    zrow = jnp.zeros((RPT, P), jnp.float32)
    iota = jnp.arange(NPAD // 16, dtype=jnp.int32).reshape(RPT // CHUNK, CHUNK)

    a0, a1, a2, a3, deg2 = _make_aggr_kernel()(
        xs[0], xs[1], xs[2], xs[3], src3, dst3, zrow, iota)
    deg = deg2.reshape(NPAD, 1)

    xp = jnp.zeros((NPAD, D), x.dtype).at[:N].set(x)
    w2 = jnp.concatenate([W2_l, W2_r], axis=0).T
    b2v = jnp.stack([jnp.zeros((), jnp.float32), b2[0]]).reshape(1, 2)
    w1lT = W1_l.T
    s = _make_dense_kernel()(
        a0, a1, a2, a3, xp, deg,
        w1lT[0 * P:1 * P], w1lT[1 * P:2 * P], w1lT[2 * P:3 * P],
        w1lT[3 * P:4 * P], W1_r.T, b1.reshape(1, H), w2, b2v)

    outf = _make_seg2_kernel()(s.reshape(NPAD * 2), src3, dst3,
                               deg.reshape(NPAD), iota)
    return outf[:N].reshape(N, 1)


# speedup vs baseline: 9.2456x; 1.0222x over previous
import functools
import jax
import jax.numpy as jnp
from jax import lax
from jax.experimental import pallas as pl
from jax.experimental.pallas import tpu as pltpu
from jax.experimental.pallas import tpu_sc as plsc

N = 10000
E = 160000
D = 256
H = 256

NC = 2
NS = 16
CHUNK = 128
EP = 163840
CPT = EP // NS // CHUNK
NPAD = 10240
RPT = NPAD // NS


def _sc_mesh():
    return plsc.VectorSubcoreMesh(core_axis_name="c", subcore_axis_name="s",
                                  num_cores=NC, num_subcores=NS)


P = 64


def _aggr_body(x00_hbm, x01_hbm, x10_hbm, x11_hbm, src_hbm, dst_hbm,
               zrow_hbm, iota_hbm,
               a0_hbm, a1_hbm, a2_hbm, a3_hbm, deg_hbm,
               aggr_sh, deg_sh, src_v, dst_v, rows0, rows1, rows2, rows3,
               deg_l, rep_v, iota_v,
               sg0, sg1, sg2, sg3, ss0, ss1, ss2, ss3):
    cid = lax.axis_index("c")
    sid = lax.axis_index("s")

    pltpu.sync_copy(src_hbm.at[sid], src_v)
    pltpu.sync_copy(dst_hbm.at[sid], dst_v)
    pltpu.sync_copy(iota_hbm, iota_v)

    zero16 = jnp.zeros((16,), jnp.float32)

    def zdeg(i, _):
        deg_l[pl.ds(i * 16, 16)] = zero16
        return 0
    lax.fori_loop(0, RPT, zdeg, 0)

    def zrep(i, _):
        rep_v[i] = zero16
        return 0
    lax.fori_loop(0, CHUNK, zrep, 0)

    @pl.when(jnp.logical_and(cid == 0, sid == 0))
    def _():
        for j in range(RPT // CHUNK):
            pltpu.sync_copy(rep_v, deg_sh.at[pl.ds(j * CHUNK, CHUNK)])

    sl = pl.ds(sid * RPT, RPT)
    ones16 = jnp.ones((16,), jnp.float32)

    def hist(c):
        for j in range(CHUNK // 16):
            d16 = dst_v[c, pl.ds(j * 16, 16)]
            plsc.addupdate_scatter(deg_l, [d16], ones16)

    bufs = [rows0, rows1, rows2, rows3]
    sgs = [sg0, sg1, sg2, sg3]
    sss = [ss0, ss1, ss2, ss3]
    NB = 4

    def run_pass(x_hbm, out_hbm, with_deg):
        def G(c, b):
            return pltpu.make_async_copy(x_hbm.at[src_v.at[c]], bufs[b],
                                         sgs[b])

        def S(c, b):
            return pltpu.make_async_copy(bufs[b], aggr_sh.at[dst_v.at[c]],
                                         sss[b])

        pltpu.sync_copy(zrow_hbm, aggr_sh.at[sl])
        plsc.subcore_barrier()
        G(0, 0).start()
        G(1, 1).start()

        def body(o, _):
            for b in range(NB):
                c = NB * o + b
                G(c, b).wait()
                S(c, b).start(add=True)
                if with_deg:
                    hist(c)
                bn = (b + 2) % NB

                @pl.when(c >= 2)
                def _():
                    S(c - 2, bn).wait()

                @pl.when(c + 2 < CPT)
                def _():
                    G(c + 2, bn).start()
            return 0
        lax.fori_loop(0, CPT // NB, body, 0)
        S(CPT - 2, (CPT - 2) % NB).wait()
        S(CPT - 1, (CPT - 1) % NB).wait()
        plsc.subcore_barrier()
        pltpu.sync_copy(aggr_sh.at[sl], out_hbm.at[sl])

    @pl.when(cid == 0)
    def _():
        run_pass(x00_hbm, a0_hbm, True)
        run_pass(x01_hbm, a1_hbm, False)

    @pl.when(cid == 1)
    def _():
        run_pass(x10_hbm, a2_hbm, False)
        run_pass(x11_hbm, a3_hbm, False)

    @pl.when(cid == 0)
    def _():
        for j in range(RPT // CHUNK):
            def rbody(i, _):
                rep_v[i] = deg_l[pl.ds(j * CHUNK * 16 + i * 16, 16)]
                return 0
            lax.fori_loop(0, CHUNK, rbody, 0)
            pltpu.sync_copy(rep_v, deg_sh.at[iota_v.at[j]], add=True)
        plsc.subcore_barrier()
        pltpu.sync_copy(deg_sh.at[pl.ds(sid * (RPT // 16), RPT // 16)],
                        deg_hbm.at[pl.ds(sid * (RPT // 16), RPT // 16)])


def _make_aggr_kernel():
    return pl.kernel(
        _aggr_body,
        out_type=(
            jax.ShapeDtypeStruct((NPAD, P), jnp.float32),
            jax.ShapeDtypeStruct((NPAD, P), jnp.float32),
            jax.ShapeDtypeStruct((NPAD, P), jnp.float32),
            jax.ShapeDtypeStruct((NPAD, P), jnp.float32),
            jax.ShapeDtypeStruct((NPAD // 16, 16), jnp.float32),
        ),
        mesh=_sc_mesh(),
        scratch_types=[
            pltpu.VMEM_SHARED((NPAD, P), jnp.float32),
            pltpu.VMEM_SHARED((NPAD // 16, 16), jnp.float32),
            pltpu.VMEM((CPT, CHUNK), jnp.int32),
            pltpu.VMEM((CPT, CHUNK), jnp.int32),
            pltpu.VMEM((CHUNK, P), jnp.float32),
            pltpu.VMEM((CHUNK, P), jnp.float32),
            pltpu.VMEM((CHUNK, P), jnp.float32),
            pltpu.VMEM((CHUNK, P), jnp.float32),
            pltpu.VMEM((NPAD,), jnp.float32),
            pltpu.VMEM((CHUNK, 16), jnp.float32),
            pltpu.VMEM((RPT // CHUNK, CHUNK), jnp.int32),
            pltpu.SemaphoreType.DMA,
            pltpu.SemaphoreType.DMA,
            pltpu.SemaphoreType.DMA,
            pltpu.SemaphoreType.DMA,
            pltpu.SemaphoreType.DMA,
            pltpu.SemaphoreType.DMA,
            pltpu.SemaphoreType.DMA,
            pltpu.SemaphoreType.DMA,
        ],
        compiler_params=pltpu.CompilerParams(needs_layout_passes=False, use_tc_tiling_on_sc=False),
    )


def _dense_body(a0_ref, a1_ref, a2_ref, a3_ref, x_ref, deg_ref,
                w1l0_ref, w1l1_ref, w1l2_ref, w1l3_ref,
                w1r_ref, b1_ref, w2_ref, b2_ref, s_ref):
    inv = 1.0 / jnp.maximum(deg_ref[...], 1.0)
    f32 = jnp.float32
    h = (jnp.dot(a0_ref[...] * inv, w1l0_ref[...], preferred_element_type=f32)
         + jnp.dot(a1_ref[...] * inv, w1l1_ref[...], preferred_element_type=f32)
         + jnp.dot(a2_ref[...] * inv, w1l2_ref[...], preferred_element_type=f32)
         + jnp.dot(a3_ref[...] * inv, w1l3_ref[...], preferred_element_type=f32)
         + jnp.dot(x_ref[...], w1r_ref[...], preferred_element_type=f32)
         + b1_ref[...])
    h = jnp.maximum(h, 0.0)
    s_ref[...] = (jnp.dot(h, w2_ref[...], preferred_element_type=f32)
                  + b2_ref[...])


def _make_dense_kernel():
    nb = NPAD // 256
    return pl.pallas_call(
        _dense_body,
        grid=(nb,),
        in_specs=[
            pl.BlockSpec((256, P), lambda i: (i, 0)),
            pl.BlockSpec((256, P), lambda i: (i, 0)),
            pl.BlockSpec((256, P), lambda i: (i, 0)),
            pl.BlockSpec((256, P), lambda i: (i, 0)),
            pl.BlockSpec((256, D), lambda i: (i, 0)),
            pl.BlockSpec((256, 1), lambda i: (i, 0)),
            pl.BlockSpec((P, H), lambda i: (0, 0)),
            pl.BlockSpec((P, H), lambda i: (0, 0)),
            pl.BlockSpec((P, H), lambda i: (0, 0)),
            pl.BlockSpec((P, H), lambda i: (0, 0)),
            pl.BlockSpec((D, H), lambda i: (0, 0)),
            pl.BlockSpec((1, H), lambda i: (0, 0)),
            pl.BlockSpec((H, 2), lambda i: (0, 0)),
            pl.BlockSpec((1, 2), lambda i: (0, 0)),
        ],
        out_specs=pl.BlockSpec((256, 2), lambda i: (i, 0)),
        out_shape=jax.ShapeDtypeStruct((NPAD, 2), jnp.float32),
    )


def _seg2_body(s_hbm, src_hbm, dst_hbm, deg_hbm, iota_hbm,
               out_hbm,
               t_sh, s_v, src_v, dst_v, deg_v, t_l, rep_v, tb_v, out_v,
               iota_v):
    cid = lax.axis_index("c")
    sid = lax.axis_index("s")

    pltpu.sync_copy(s_hbm, s_v)
    pltpu.sync_copy(src_hbm.at[sid], src_v)
    pltpu.sync_copy(dst_hbm.at[sid], dst_v)
    pltpu.sync_copy(deg_hbm, deg_v)
    pltpu.sync_copy(iota_hbm, iota_v)

    zero16 = jnp.zeros((16,), jnp.float32)

    def zt(i, _):
        t_l[pl.ds(i * 16, 16)] = zero16
        return 0
    lax.fori_loop(0, RPT, zt, 0)

    def zrep(i, _):
        rep_v[i] = zero16
        return 0
    lax.fori_loop(0, CHUNK, zrep, 0)

    @pl.when(sid == 0)
    def _():
        for j in range(RPT // CHUNK):
            pltpu.sync_copy(rep_v, t_sh.at[pl.ds(j * CHUNK, CHUNK)])

    plsc.subcore_barrier()

    def body(c, _):
        for j in range(CHUNK // 16):
            s16 = src_v[c, pl.ds(j * 16, 16)]
            d16 = dst_v[c, pl.ds(j * 16, 16)]
            v = plsc.load_gather(s_v, [lax.shift_left(s16, 1)])
            plsc.addupdate_scatter(t_l, [d16], v)
        return 0
    lax.fori_loop(0, CPT, body, 0)

    for j in range(RPT // CHUNK):
        def rbody(i, _):
            rep_v[i] = t_l[pl.ds(j * CHUNK * 16 + i * 16, 16)]
            return 0
        lax.fori_loop(0, CHUNK, rbody, 0)
        pltpu.sync_copy(rep_v, t_sh.at[iota_v.at[j]], add=True)

    plsc.subcore_barrier()

    @pl.when(cid == 0)
    def _():
        pltpu.sync_copy(t_sh.at[pl.ds(sid * (RPT // 16), RPT // 16)], tb_v)
        base = sid * RPT
        i16 = lax.iota(jnp.int32, 16)

        def fbody(j, _):
            tt = tb_v[j]
            dd = deg_v[pl.ds(base + j * 16, 16)]
            s1 = plsc.load_gather(
                s_v, [lax.shift_left(i16 + (base + j * 16), 1) + 1])
            out_v[pl.ds(j * 16, 16)] = tt / jnp.maximum(dd, 1.0) + s1
            return 0
        lax.fori_loop(0, RPT // 16, fbody, 0)
        pltpu.sync_copy(out_v, out_hbm.at[pl.ds(base, RPT)])


def _make_seg2_kernel():
    return pl.kernel(
        _seg2_body,
        out_type=jax.ShapeDtypeStruct((NPAD,), jnp.float32),
        mesh=_sc_mesh(),
        scratch_types=[
            pltpu.VMEM_SHARED((NPAD // 16, 16), jnp.float32),
            pltpu.VMEM((NPAD * 2,), jnp.float32),
            pltpu.VMEM((CPT, CHUNK), jnp.int32),
            pltpu.VMEM((CPT, CHUNK), jnp.int32),
            pltpu.VMEM((NPAD,), jnp.float32),
            pltpu.VMEM((NPAD,), jnp.float32),
            pltpu.VMEM((CHUNK, 16), jnp.float32),
            pltpu.VMEM((RPT // 16, 16), jnp.float32),
            pltpu.VMEM((RPT,), jnp.float32),
            pltpu.VMEM((RPT // CHUNK, CHUNK), jnp.int32),
        ],
        compiler_params=pltpu.CompilerParams(needs_layout_passes=False, use_tc_tiling_on_sc=False),
    )


@jax.jit
def kernel(x, edge_index, W1_l, W1_r, b1, W2_l, W2_r, b2):
    npad = EP - E
    pad_src = (jnp.arange(npad, dtype=jnp.int32) * 37) % N
    pad_dst = N + (jnp.arange(npad, dtype=jnp.int32) % (NPAD - N))
    src = jnp.concatenate([edge_index[0].astype(jnp.int32), pad_src])
    dst = jnp.concatenate([edge_index[1].astype(jnp.int32), pad_dst])
    src3 = src.reshape(NS, CPT, CHUNK)
    dst3 = dst.reshape(NS, CPT, CHUNK)

    xs = [x[:, i * P:(i + 1) * P] for i in range(4)]
    zrow = jnp.zeros((RPT, P), jnp.float32)
    iota = jnp.arange(NPAD // 16, dtype=jnp.int32).reshape(RPT // CHUNK, CHUNK)

    a0, a1, a2, a3, deg2 = _make_aggr_kernel()(
        xs[0], xs[1], xs[2], xs[3], src3, dst3, zrow, iota)
    deg = deg2.reshape(NPAD, 1)

    xp = jnp.zeros((NPAD, D), x.dtype).at[:N].set(x)
    w2 = jnp.concatenate([W2_l, W2_r], axis=0).T
    b2v = jnp.stack([jnp.zeros((), jnp.float32), b2[0]]).reshape(1, 2)
    w1lT = W1_l.T
    s = _make_dense_kernel()(
        a0, a1, a2, a3, xp, deg,
        w1lT[0 * P:1 * P], w1lT[1 * P:2 * P], w1lT[2 * P:3 * P],
        w1lT[3 * P:4 * P], W1_r.T, b1.reshape(1, H), w2, b2v)

    outf = _make_seg2_kernel()(s.reshape(NPAD * 2), src3, dst3,
                               deg.reshape(NPAD), iota)
    return outf[:N].reshape(N, 1)


# speedup vs baseline: 11.7420x; 1.2700x over previous
import functools
import jax
import jax.numpy as jnp
from jax import lax
from jax.experimental import pallas as pl
from jax.experimental.pallas import tpu as pltpu
from jax.experimental.pallas import tpu_sc as plsc

N = 10000
E = 160000
D = 256
H = 256

NC = 2
NS = 16
CHUNK = 128
EP = 163840
CPT = EP // NS // CHUNK
NPAD = 10240
RPT = NPAD // NS


def _sc_mesh():
    return plsc.VectorSubcoreMesh(core_axis_name="c", subcore_axis_name="s",
                                  num_cores=NC, num_subcores=NS)


P = 64


def _aggr_body(xf_hbm, src_hbm, dst_hbm, zrow_hbm, iota_hbm,
               aL_hbm, aR_hbm, deg_hbm,
               aggr_sh, deg_sh, src_v, dst_v, rows0, rows1, rows2, rows3,
               deg_l, rep_v, iota_v,
               sg0, sg1, sg2, sg3, ss0, ss1, ss2, ss3):
    cid = lax.axis_index("c")
    sid = lax.axis_index("s")

    pltpu.sync_copy(src_hbm.at[sid], src_v)
    pltpu.sync_copy(dst_hbm.at[sid], dst_v)
    pltpu.sync_copy(iota_hbm, iota_v)

    zero16 = jnp.zeros((16,), jnp.float32)

    def zdeg(i, _):
        deg_l[pl.ds(i * 16, 16)] = zero16
        return 0
    lax.fori_loop(0, RPT, zdeg, 0)

    def zrep(i, _):
        rep_v[i] = zero16
        return 0
    lax.fori_loop(0, CHUNK, zrep, 0)

    @pl.when(jnp.logical_and(cid == 0, sid == 0))
    def _():
        for j in range(RPT // CHUNK):
            pltpu.sync_copy(rep_v, deg_sh.at[pl.ds(j * CHUNK, CHUNK)])

    sl = pl.ds(sid * RPT, RPT)
    ones16 = jnp.ones((16,), jnp.float32)

    def hist(c):
        for j in range(CHUNK // 16):
            d16 = dst_v[c, pl.ds(j * 16, 16)]
            plsc.addupdate_scatter(deg_l, [d16], ones16)

    bufs = [rows0, rows1, rows2, rows3]
    sgs = [sg0, sg1, sg2, sg3]
    sss = [ss0, ss1, ss2, ss3]
    NB = 4

    def bump(delta):
        d16 = jnp.zeros((16,), jnp.int32) + delta

        def bb(c, _):
            for j in range(CHUNK // 16):
                sl2 = pl.ds(j * 16, 16)
                src_v[c, sl2] = src_v[c, sl2] + d16
            return 0
        lax.fori_loop(0, CPT, bb, 0)

    def run_pass(out_hbm, col_off, with_deg):
        def G(c, b):
            return pltpu.make_async_copy(xf_hbm.at[src_v.at[c]], bufs[b],
                                         sgs[b])

        def S(c, b):
            return pltpu.make_async_copy(bufs[b], aggr_sh.at[dst_v.at[c]],
                                         sss[b])

        pltpu.sync_copy(zrow_hbm, aggr_sh.at[sl])
        plsc.subcore_barrier()
        G(0, 0).start()
        G(1, 1).start()

        def body(o, _):
            for b in range(NB):
                c = NB * o + b
                G(c, b).wait()
                S(c, b).start(add=True)
                if with_deg:
                    hist(c)
                bn = (b + 2) % NB

                @pl.when(c >= 2)
                def _():
                    S(c - 2, bn).wait()

                @pl.when(c + 2 < CPT)
                def _():
                    G(c + 2, bn).start()
            return 0
        lax.fori_loop(0, CPT // NB, body, 0)
        S(CPT - 2, (CPT - 2) % NB).wait()
        S(CPT - 1, (CPT - 1) % NB).wait()
        plsc.subcore_barrier()
        pltpu.sync_copy(aggr_sh.at[sl],
                        out_hbm.at[sl, pl.ds(col_off, P)])

    bump(2 * cid)

    @pl.when(cid == 0)
    def _():
        run_pass(aL_hbm, 0, True)
        bump(1)
        run_pass(aL_hbm, P, False)

    @pl.when(cid == 1)
    def _():
        run_pass(aR_hbm, 0, False)
        bump(1)
        run_pass(aR_hbm, P, False)

    @pl.when(cid == 0)
    def _():
        for j in range(RPT // CHUNK):
            def rbody(i, _):
                rep_v[i] = deg_l[pl.ds(j * CHUNK * 16 + i * 16, 16)]
                return 0
            lax.fori_loop(0, CHUNK, rbody, 0)
            pltpu.sync_copy(rep_v, deg_sh.at[iota_v.at[j]], add=True)
        plsc.subcore_barrier()
        pltpu.sync_copy(deg_sh.at[pl.ds(sid * (RPT // 16), RPT // 16)],
                        deg_hbm.at[pl.ds(sid * (RPT // 16), RPT // 16)])


def _make_aggr_kernel():
    return pl.kernel(
        _aggr_body,
        out_type=(
            jax.ShapeDtypeStruct((NPAD, 2 * P), jnp.float32),
            jax.ShapeDtypeStruct((NPAD, 2 * P), jnp.float32),
            jax.ShapeDtypeStruct((NPAD // 16, 16), jnp.float32),
        ),
        mesh=_sc_mesh(),
        scratch_types=[
            pltpu.VMEM_SHARED((NPAD, P), jnp.float32),
            pltpu.VMEM_SHARED((NPAD // 16, 16), jnp.float32),
            pltpu.VMEM((CPT, CHUNK), jnp.int32),
            pltpu.VMEM((CPT, CHUNK), jnp.int32),
            pltpu.VMEM((CHUNK, P), jnp.float32),
            pltpu.VMEM((CHUNK, P), jnp.float32),
            pltpu.VMEM((CHUNK, P), jnp.float32),
            pltpu.VMEM((CHUNK, P), jnp.float32),
            pltpu.VMEM((NPAD,), jnp.float32),
            pltpu.VMEM((CHUNK, 16), jnp.float32),
            pltpu.VMEM((RPT // CHUNK, CHUNK), jnp.int32),
            pltpu.SemaphoreType.DMA,
            pltpu.SemaphoreType.DMA,
            pltpu.SemaphoreType.DMA,
            pltpu.SemaphoreType.DMA,
            pltpu.SemaphoreType.DMA,
            pltpu.SemaphoreType.DMA,
            pltpu.SemaphoreType.DMA,
            pltpu.SemaphoreType.DMA,
        ],
        compiler_params=pltpu.CompilerParams(needs_layout_passes=False, use_tc_tiling_on_sc=False),
    )


BN = 512


def _dense_body(aL_ref, aR_ref, x_ref, deg_ref, w1l0_ref, w1l1_ref,
                w1r_ref, b1_ref, w2_ref, b2_ref, s_ref):
    inv = 1.0 / jnp.maximum(deg_ref[...], 1.0)
    f32 = jnp.float32
    h = (jnp.dot(aL_ref[...] * inv, w1l0_ref[...], preferred_element_type=f32)
         + jnp.dot(aR_ref[...] * inv, w1l1_ref[...], preferred_element_type=f32)
         + jnp.dot(x_ref[...], w1r_ref[...], preferred_element_type=f32)
         + b1_ref[...])
    h = jnp.maximum(h, 0.0)
    s_ref[...] = (jnp.dot(h, w2_ref[...], preferred_element_type=f32)
                  + b2_ref[...])


def _make_dense_kernel():
    nb = NPAD // BN
    return pl.pallas_call(
        _dense_body,
        grid=(nb,),
        in_specs=[
            pl.BlockSpec((BN, 2 * P), lambda i: (i, 0)),
            pl.BlockSpec((BN, 2 * P), lambda i: (i, 0)),
            pl.BlockSpec((BN, D), lambda i: (i, 0)),
            pl.BlockSpec((BN, 1), lambda i: (i, 0)),
            pl.BlockSpec((2 * P, H), lambda i: (0, 0)),
            pl.BlockSpec((2 * P, H), lambda i: (0, 0)),
            pl.BlockSpec((D, H), lambda i: (0, 0)),
            pl.BlockSpec((1, H), lambda i: (0, 0)),
            pl.BlockSpec((H, 2), lambda i: (0, 0)),
            pl.BlockSpec((1, 2), lambda i: (0, 0)),
        ],
        out_specs=pl.BlockSpec((BN, 2), lambda i: (i, 0)),
        out_shape=jax.ShapeDtypeStruct((NPAD, 2), jnp.float32),
    )


def _seg2_body(s_hbm, src_hbm, dst_hbm, deg_hbm, iota_hbm,
               out_hbm,
               t_sh, s_v, src_v, dst_v, deg_v, t_l, rep_v, tb_v, out_v,
               iota_v):
    cid = lax.axis_index("c")
    sid = lax.axis_index("s")

    pltpu.sync_copy(s_hbm, s_v)
    pltpu.sync_copy(src_hbm.at[sid], src_v)
    pltpu.sync_copy(dst_hbm.at[sid], dst_v)
    pltpu.sync_copy(deg_hbm, deg_v)
    pltpu.sync_copy(iota_hbm, iota_v)

    zero16 = jnp.zeros((16,), jnp.float32)

    def zt(i, _):
        t_l[pl.ds(i * 16, 16)] = zero16
        return 0
    lax.fori_loop(0, RPT, zt, 0)

    def zrep(i, _):
        rep_v[i] = zero16
        return 0
    lax.fori_loop(0, CHUNK, zrep, 0)

    @pl.when(sid == 0)
    def _():
        for j in range(RPT // CHUNK):
            pltpu.sync_copy(rep_v, t_sh.at[pl.ds(j * CHUNK, CHUNK)])

    plsc.subcore_barrier()

    def body(c, _):
        for j in range(CHUNK // 16):
            s16 = src_v[c, pl.ds(j * 16, 16)]
            d16 = dst_v[c, pl.ds(j * 16, 16)]
            v = plsc.load_gather(s_v, [lax.shift_right_logical(s16, 1)])
            plsc.addupdate_scatter(t_l, [d16], v)
        return 0
    lax.fori_loop(0, CPT, body, 0)

    for j in range(RPT // CHUNK):
        def rbody(i, _):
            rep_v[i] = t_l[pl.ds(j * CHUNK * 16 + i * 16, 16)]
            return 0
        lax.fori_loop(0, CHUNK, rbody, 0)
        pltpu.sync_copy(rep_v, t_sh.at[iota_v.at[j]], add=True)

    plsc.subcore_barrier()

    @pl.when(cid == 0)
    def _():
        pltpu.sync_copy(t_sh.at[pl.ds(sid * (RPT // 16), RPT // 16)], tb_v)
        base = sid * RPT
        i16 = lax.iota(jnp.int32, 16)

        def fbody(j, _):
            tt = tb_v[j]
            dd = deg_v[pl.ds(base + j * 16, 16)]
            s1 = plsc.load_gather(
                s_v, [lax.shift_left(i16 + (base + j * 16), 1) + 1])
            out_v[pl.ds(j * 16, 16)] = tt / jnp.maximum(dd, 1.0) + s1
            return 0
        lax.fori_loop(0, RPT // 16, fbody, 0)
        pltpu.sync_copy(out_v, out_hbm.at[pl.ds(base, RPT)])


def _make_seg2_kernel():
    return pl.kernel(
        _seg2_body,
        out_type=jax.ShapeDtypeStruct((NPAD,), jnp.float32),
        mesh=_sc_mesh(),
        scratch_types=[
            pltpu.VMEM_SHARED((NPAD // 16, 16), jnp.float32),
            pltpu.VMEM((NPAD * 2,), jnp.float32),
            pltpu.VMEM((CPT, CHUNK), jnp.int32),
            pltpu.VMEM((CPT, CHUNK), jnp.int32),
            pltpu.VMEM((NPAD,), jnp.float32),
            pltpu.VMEM((NPAD,), jnp.float32),
            pltpu.VMEM((CHUNK, 16), jnp.float32),
            pltpu.VMEM((RPT // 16, 16), jnp.float32),
            pltpu.VMEM((RPT,), jnp.float32),
            pltpu.VMEM((RPT // CHUNK, CHUNK), jnp.int32),
        ],
        compiler_params=pltpu.CompilerParams(needs_layout_passes=False, use_tc_tiling_on_sc=False),
    )


@jax.jit
def kernel(x, edge_index, W1_l, W1_r, b1, W2_l, W2_r, b2):
    npad = EP - E
    pad_src = (jnp.arange(npad, dtype=jnp.int32) * 37) % N
    pad_dst = N + (jnp.arange(npad, dtype=jnp.int32) % (NPAD - N))
    src = jnp.concatenate([edge_index[0].astype(jnp.int32), pad_src]) * 4
    dst = jnp.concatenate([edge_index[1].astype(jnp.int32), pad_dst])
    src3 = src.reshape(NS, CPT, CHUNK)
    dst3 = dst.reshape(NS, CPT, CHUNK)

    xf = x.reshape(4 * N, P)
    zrow = jnp.zeros((RPT, P), jnp.float32)
    iota = jnp.arange(NPAD // 16, dtype=jnp.int32).reshape(RPT // CHUNK, CHUNK)

    aL, aR, deg2 = _make_aggr_kernel()(xf, src3, dst3, zrow, iota)
    deg = deg2.reshape(NPAD, 1)

    w2 = jnp.concatenate([W2_l, W2_r], axis=0).T
    b2v = jnp.stack([jnp.zeros((), jnp.float32), b2[0]]).reshape(1, 2)
    w1lT = W1_l.T
    s = _make_dense_kernel()(
        aL, aR, x, deg, w1lT[:2 * P], w1lT[2 * P:], W1_r.T,
        b1.reshape(1, H), w2, b2v)

    outf = _make_seg2_kernel()(s.reshape(NPAD * 2), src3, dst3,
                               deg.reshape(NPAD), iota)
    return outf[:N].reshape(N, 1)


# speedup vs baseline: 13.7662x; 1.1724x over previous
import functools
import jax
import jax.numpy as jnp
from jax import lax
from jax.experimental import pallas as pl
from jax.experimental.pallas import tpu as pltpu
from jax.experimental.pallas import tpu_sc as plsc

N = 10000
E = 160000
D = 256
H = 256

NC = 2
NS = 16
CHUNK = 128
EP = 163840
CPT = EP // NS // CHUNK
NPAD = 10240
RPT = NPAD // NS


def _sc_mesh():
    return plsc.VectorSubcoreMesh(core_axis_name="c", subcore_axis_name="s",
                                  num_cores=NC, num_subcores=NS)


P = 64


def _aggr_body(xf_hbm, src_hbm, dst_hbm, zrow_hbm, iota_hbm,
               aL_hbm, aR_hbm, deg_hbm,
               aggr_sh, deg_sh, src_v, dst_v, rows0, rows1, rows2, rows3,
               deg_l, rep_v, iota_v,
               sg0, sg1, sg2, sg3, ss0, ss1, ss2, ss3):
    cid = lax.axis_index("c")
    sid = lax.axis_index("s")

    pltpu.sync_copy(src_hbm.at[sid], src_v)
    pltpu.sync_copy(dst_hbm.at[sid], dst_v)
    pltpu.sync_copy(iota_hbm, iota_v)

    zero16 = jnp.zeros((16,), jnp.float32)

    def zdeg(i, _):
        deg_l[pl.ds(i * 16, 16)] = zero16
        return 0
    lax.fori_loop(0, RPT, zdeg, 0)

    def zrep(i, _):
        rep_v[i] = zero16
        return 0
    lax.fori_loop(0, CHUNK, zrep, 0)

    @pl.when(jnp.logical_and(cid == 0, sid == 0))
    def _():
        for j in range(RPT // CHUNK):
            pltpu.sync_copy(rep_v, deg_sh.at[pl.ds(j * CHUNK, CHUNK)])

    sl = pl.ds(sid * RPT, RPT)
    ones16 = jnp.ones((16,), jnp.float32)

    def hist(c):
        for j in range(CHUNK // 16):
            d16 = dst_v[c, pl.ds(j * 16, 16)]
            plsc.addupdate_scatter(deg_l, [d16], ones16)

    bufs = [rows0, rows1, rows2, rows3]
    sgs = [sg0, sg1, sg2, sg3]
    sss = [ss0, ss1, ss2, ss3]
    NB = 4

    def bump(delta):
        d16 = jnp.zeros((16,), jnp.int32) + delta

        def bb(c, _):
            for j in range(CHUNK // 16):
                sl2 = pl.ds(j * 16, 16)
                src_v[c, sl2] = src_v[c, sl2] + d16
            return 0
        lax.fori_loop(0, CPT, bb, 0)

    def run_pass(out_hbm, with_deg):
        def G(c, b):
            return pltpu.make_async_copy(xf_hbm.at[src_v.at[c]], bufs[b],
                                         sgs[b])

        def S(c, b):
            return pltpu.make_async_copy(bufs[b], aggr_sh.at[dst_v.at[c]],
                                         sss[b])

        pltpu.sync_copy(zrow_hbm, aggr_sh.at[sl])
        plsc.subcore_barrier()
        G(0, 0).start()
        G(1, 1).start()

        def body(o, _):
            for b in range(NB):
                c = NB * o + b
                G(c, b).wait()
                S(c, b).start(add=True)
                if with_deg:
                    hist(c)
                bn = (b + 2) % NB

                @pl.when(c >= 2)
                def _():
                    S(c - 2, bn).wait()

                @pl.when(c + 2 < CPT)
                def _():
                    G(c + 2, bn).start()
            return 0
        lax.fori_loop(0, CPT // NB, body, 0)
        S(CPT - 2, (CPT - 2) % NB).wait()
        S(CPT - 1, (CPT - 1) % NB).wait()
        plsc.subcore_barrier()
        pltpu.sync_copy(aggr_sh.at[sl], out_hbm.at[sl])

    bump(cid)

    @pl.when(cid == 0)
    def _():
        run_pass(aL_hbm, True)

    @pl.when(cid == 1)
    def _():
        run_pass(aR_hbm, False)

    @pl.when(cid == 0)
    def _():
        for j in range(RPT // CHUNK):
            def rbody(i, _):
                rep_v[i] = deg_l[pl.ds(j * CHUNK * 16 + i * 16, 16)]
                return 0
            lax.fori_loop(0, CHUNK, rbody, 0)
            pltpu.sync_copy(rep_v, deg_sh.at[iota_v.at[j]], add=True)
        plsc.subcore_barrier()
        pltpu.sync_copy(deg_sh.at[pl.ds(sid * (RPT // 16), RPT // 16)],
                        deg_hbm.at[pl.ds(sid * (RPT // 16), RPT // 16)])


def _make_aggr_kernel():
    return pl.kernel(
        _aggr_body,
        out_type=(
            jax.ShapeDtypeStruct((NPAD, 2 * P), jnp.bfloat16),
            jax.ShapeDtypeStruct((NPAD, 2 * P), jnp.bfloat16),
            jax.ShapeDtypeStruct((NPAD // 16, 16), jnp.float32),
        ),
        mesh=_sc_mesh(),
        scratch_types=[
            pltpu.VMEM_SHARED((NPAD, 2 * P), jnp.bfloat16),
            pltpu.VMEM_SHARED((NPAD // 16, 16), jnp.float32),
            pltpu.VMEM((CPT, CHUNK), jnp.int32),
            pltpu.VMEM((CPT, CHUNK), jnp.int32),
            pltpu.VMEM((CHUNK, 2 * P), jnp.bfloat16),
            pltpu.VMEM((CHUNK, 2 * P), jnp.bfloat16),
            pltpu.VMEM((CHUNK, 2 * P), jnp.bfloat16),
            pltpu.VMEM((CHUNK, 2 * P), jnp.bfloat16),
            pltpu.VMEM((NPAD,), jnp.float32),
            pltpu.VMEM((CHUNK, 16), jnp.float32),
            pltpu.VMEM((RPT // CHUNK, CHUNK), jnp.int32),
            pltpu.SemaphoreType.DMA,
            pltpu.SemaphoreType.DMA,
            pltpu.SemaphoreType.DMA,
            pltpu.SemaphoreType.DMA,
            pltpu.SemaphoreType.DMA,
            pltpu.SemaphoreType.DMA,
            pltpu.SemaphoreType.DMA,
            pltpu.SemaphoreType.DMA,
        ],
        compiler_params=pltpu.CompilerParams(needs_layout_passes=False, use_tc_tiling_on_sc=False),
    )


BN = 512


def _dense_body(aL_ref, aR_ref, x_ref, deg_ref, w1l0_ref, w1l1_ref,
                w1r_ref, b1_ref, w2_ref, b2_ref, s_ref):
    inv = 1.0 / jnp.maximum(deg_ref[...], 1.0)
    f32 = jnp.float32
    aL = aL_ref[...].astype(f32) * inv
    aR = aR_ref[...].astype(f32) * inv
    h = (jnp.dot(aL, w1l0_ref[...], preferred_element_type=f32)
         + jnp.dot(aR, w1l1_ref[...], preferred_element_type=f32)
         + jnp.dot(x_ref[...], w1r_ref[...], preferred_element_type=f32)
         + b1_ref[...])
    h = jnp.maximum(h, 0.0)
    s_ref[...] = (jnp.dot(h, w2_ref[...], preferred_element_type=f32)
                  + b2_ref[...])


def _make_dense_kernel():
    nb = NPAD // BN
    return pl.pallas_call(
        _dense_body,
        grid=(nb,),
        in_specs=[
            pl.BlockSpec((BN, 2 * P), lambda i: (i, 0)),
            pl.BlockSpec((BN, 2 * P), lambda i: (i, 0)),
            pl.BlockSpec((BN, D), lambda i: (i, 0)),
            pl.BlockSpec((BN, 1), lambda i: (i, 0)),
            pl.BlockSpec((2 * P, H), lambda i: (0, 0)),
            pl.BlockSpec((2 * P, H), lambda i: (0, 0)),
            pl.BlockSpec((D, H), lambda i: (0, 0)),
            pl.BlockSpec((1, H), lambda i: (0, 0)),
            pl.BlockSpec((H, 2), lambda i: (0, 0)),
            pl.BlockSpec((1, 2), lambda i: (0, 0)),
        ],
        out_specs=pl.BlockSpec((BN, 2), lambda i: (i, 0)),
        out_shape=jax.ShapeDtypeStruct((NPAD, 2), jnp.float32),
    )


def _seg2_body(s_hbm, src_hbm, dst_hbm, deg_hbm, iota_hbm,
               out_hbm,
               t_sh, s_v, src_v, dst_v, deg_v, t_l, rep_v, tb_v, out_v,
               iota_v):
    cid = lax.axis_index("c")
    sid = lax.axis_index("s")

    pltpu.sync_copy(s_hbm, s_v)
    pltpu.sync_copy(src_hbm.at[sid], src_v)
    pltpu.sync_copy(dst_hbm.at[sid], dst_v)
    pltpu.sync_copy(deg_hbm, deg_v)
    pltpu.sync_copy(iota_hbm, iota_v)

    zero16 = jnp.zeros((16,), jnp.float32)

    def zt(i, _):
        t_l[pl.ds(i * 16, 16)] = zero16
        return 0
    lax.fori_loop(0, RPT, zt, 0)

    def zrep(i, _):
        rep_v[i] = zero16
        return 0
    lax.fori_loop(0, CHUNK, zrep, 0)

    @pl.when(sid == 0)
    def _():
        for j in range(RPT // CHUNK):
            pltpu.sync_copy(rep_v, t_sh.at[pl.ds(j * CHUNK, CHUNK)])

    plsc.subcore_barrier()

    def body(c, _):
        for j in range(CHUNK // 16):
            s16 = src_v[c, pl.ds(j * 16, 16)]
            d16 = dst_v[c, pl.ds(j * 16, 16)]
            v = plsc.load_gather(s_v, [s16])
            plsc.addupdate_scatter(t_l, [d16], v)
        return 0
    lax.fori_loop(0, CPT, body, 0)

    for j in range(RPT // CHUNK):
        def rbody(i, _):
            rep_v[i] = t_l[pl.ds(j * CHUNK * 16 + i * 16, 16)]
            return 0
        lax.fori_loop(0, CHUNK, rbody, 0)
        pltpu.sync_copy(rep_v, t_sh.at[iota_v.at[j]], add=True)

    plsc.subcore_barrier()

    @pl.when(cid == 0)
    def _():
        pltpu.sync_copy(t_sh.at[pl.ds(sid * (RPT // 16), RPT // 16)], tb_v)
        base = sid * RPT
        i16 = lax.iota(jnp.int32, 16)

        def fbody(j, _):
            tt = tb_v[j]
            dd = deg_v[pl.ds(base + j * 16, 16)]
            s1 = plsc.load_gather(
                s_v, [lax.shift_left(i16 + (base + j * 16), 1) + 1])
            out_v[pl.ds(j * 16, 16)] = tt / jnp.maximum(dd, 1.0) + s1
            return 0
        lax.fori_loop(0, RPT // 16, fbody, 0)
        pltpu.sync_copy(out_v, out_hbm.at[pl.ds(base, RPT)])


def _make_seg2_kernel():
    return pl.kernel(
        _seg2_body,
        out_type=jax.ShapeDtypeStruct((NPAD,), jnp.float32),
        mesh=_sc_mesh(),
        scratch_types=[
            pltpu.VMEM_SHARED((NPAD // 16, 16), jnp.float32),
            pltpu.VMEM((NPAD * 2,), jnp.float32),
            pltpu.VMEM((CPT, CHUNK), jnp.int32),
            pltpu.VMEM((CPT, CHUNK), jnp.int32),
            pltpu.VMEM((NPAD,), jnp.float32),
            pltpu.VMEM((NPAD,), jnp.float32),
            pltpu.VMEM((CHUNK, 16), jnp.float32),
            pltpu.VMEM((RPT // 16, 16), jnp.float32),
            pltpu.VMEM((RPT,), jnp.float32),
            pltpu.VMEM((RPT // CHUNK, CHUNK), jnp.int32),
        ],
        compiler_params=pltpu.CompilerParams(needs_layout_passes=False, use_tc_tiling_on_sc=False),
    )


@jax.jit
def kernel(x, edge_index, W1_l, W1_r, b1, W2_l, W2_r, b2):
    npad = EP - E
    pad_src = (jnp.arange(npad, dtype=jnp.int32) * 37) % N
    pad_dst = N + (jnp.arange(npad, dtype=jnp.int32) % (NPAD - N))
    src = jnp.concatenate([edge_index[0].astype(jnp.int32), pad_src]) * 2
    dst = jnp.concatenate([edge_index[1].astype(jnp.int32), pad_dst])
    src3 = src.reshape(NS, CPT, CHUNK)
    dst3 = dst.reshape(NS, CPT, CHUNK)

    xf = x.astype(jnp.bfloat16).reshape(2 * N, 2 * P)
    zrow = jnp.zeros((RPT, 2 * P), jnp.bfloat16)
    iota = jnp.arange(NPAD // 16, dtype=jnp.int32).reshape(RPT // CHUNK, CHUNK)

    aL, aR, deg2 = _make_aggr_kernel()(xf, src3, dst3, zrow, iota)
    deg = deg2.reshape(NPAD, 1)

    w2 = jnp.concatenate([W2_l, W2_r], axis=0).T
    b2v = jnp.stack([jnp.zeros((), jnp.float32), b2[0]]).reshape(1, 2)
    w1lT = W1_l.T
    s = _make_dense_kernel()(
        aL, aR, x, deg, w1lT[:2 * P], w1lT[2 * P:], W1_r.T,
        b1.reshape(1, H), w2, b2v)

    outf = _make_seg2_kernel()(s.reshape(NPAD * 2), src3, dst3,
                               deg.reshape(NPAD), iota)
    return outf[:N].reshape(N, 1)


# speedup vs baseline: 14.3650x; 1.0435x over previous
import functools
import jax
import jax.numpy as jnp
from jax import lax
from jax.experimental import pallas as pl
from jax.experimental.pallas import tpu as pltpu
from jax.experimental.pallas import tpu_sc as plsc

N = 10000
E = 160000
D = 256
H = 256

NC = 2
NS = 16
CHUNK = 128
EP = 163840
CPT = EP // NS // CHUNK
NPAD = 10240
RPT = NPAD // NS
CA = 256
CPTA = EP // NS // CA


def _sc_mesh():
    return plsc.VectorSubcoreMesh(core_axis_name="c", subcore_axis_name="s",
                                  num_cores=NC, num_subcores=NS)


P = 64


def _aggr_body(xf_hbm, src_hbm, dst_hbm, zrow_hbm, iota_hbm,
               aL_hbm, aR_hbm, deg_hbm,
               aggr_sh, deg_sh, src_v, dst_v, rows0, rows1,
               deg_l, rep_v, iota_v, sg0, sg1):
    cid = lax.axis_index("c")
    sid = lax.axis_index("s")

    pltpu.sync_copy(src_hbm.at[sid], src_v)
    pltpu.sync_copy(dst_hbm.at[sid], dst_v)
    pltpu.sync_copy(iota_hbm, iota_v)

    zero16 = jnp.zeros((16,), jnp.float32)

    def zdeg(i, _):
        deg_l[pl.ds(i * 16, 16)] = zero16
        return 0
    lax.fori_loop(0, RPT, zdeg, 0)

    def zrep(i, _):
        rep_v[i] = zero16
        return 0
    lax.fori_loop(0, CHUNK, zrep, 0)

    @pl.when(jnp.logical_and(cid == 0, sid == 0))
    def _():
        for j in range(RPT // CHUNK):
            pltpu.sync_copy(rep_v, deg_sh.at[pl.ds(j * CHUNK, CHUNK)])

    sl = pl.ds(sid * RPT, RPT)
    ones16 = jnp.ones((16,), jnp.float32)

    def hist(c):
        for j in range(CA // 16):
            d16 = dst_v[c, pl.ds(j * 16, 16)]
            plsc.addupdate_scatter(deg_l, [d16], ones16)

    def bump(delta):
        d16 = jnp.zeros((16,), jnp.int32) + delta

        def bb(c, _):
            for j in range(CA // 16):
                sl2 = pl.ds(j * 16, 16)
                src_v[c, sl2] = src_v[c, sl2] + d16
            return 0
        lax.fori_loop(0, CPTA, bb, 0)

    def run_pass(out_hbm, with_deg):
        def G(c, buf, sem):
            return pltpu.make_async_copy(xf_hbm.at[src_v.at[c]], buf, sem)

        pltpu.sync_copy(zrow_hbm, aggr_sh.at[sl])
        plsc.subcore_barrier()
        G(0, rows0, sg0).start()

        def body(o, _):
            c0 = 2 * o
            G(c0 + 1, rows1, sg1).start()
            if with_deg:
                hist(c0)
            G(c0, rows0, sg0).wait()
            pltpu.sync_copy(rows0, aggr_sh.at[dst_v.at[c0]], add=True)

            @pl.when(c0 + 2 < CPTA)
            def _():
                G(c0 + 2, rows0, sg0).start()
            if with_deg:
                hist(c0 + 1)
            G(c0 + 1, rows1, sg1).wait()
            pltpu.sync_copy(rows1, aggr_sh.at[dst_v.at[c0 + 1]], add=True)
            return 0
        lax.fori_loop(0, CPTA // 2, body, 0)
        plsc.subcore_barrier()
        pltpu.sync_copy(aggr_sh.at[sl], out_hbm.at[sl])

    bump(cid)

    @pl.when(cid == 0)
    def _():
        run_pass(aL_hbm, True)

    @pl.when(cid == 1)
    def _():
        run_pass(aR_hbm, False)

    @pl.when(cid == 0)
    def _():
        for j in range(RPT // CHUNK):
            def rbody(i, _):
                rep_v[i] = deg_l[pl.ds(j * CHUNK * 16 + i * 16, 16)]
                return 0
            lax.fori_loop(0, CHUNK, rbody, 0)
            pltpu.sync_copy(rep_v, deg_sh.at[iota_v.at[j]], add=True)
        plsc.subcore_barrier()
        pltpu.sync_copy(deg_sh.at[pl.ds(sid * (RPT // 16), RPT // 16)],
                        deg_hbm.at[pl.ds(sid * (RPT // 16), RPT // 16)])


def _make_aggr_kernel():
    return pl.kernel(
        _aggr_body,
        out_type=(
            jax.ShapeDtypeStruct((NPAD, 2 * P), jnp.bfloat16),
            jax.ShapeDtypeStruct((NPAD, 2 * P), jnp.bfloat16),
            jax.ShapeDtypeStruct((NPAD // 16, 16), jnp.float32),
        ),
        mesh=_sc_mesh(),
        scratch_types=[
            pltpu.VMEM_SHARED((NPAD, 2 * P), jnp.bfloat16),
            pltpu.VMEM_SHARED((NPAD // 16, 16), jnp.float32),
            pltpu.VMEM((CPTA, CA), jnp.int32),
            pltpu.VMEM((CPTA, CA), jnp.int32),
            pltpu.VMEM((CA, 2 * P), jnp.bfloat16),
            pltpu.VMEM((CA, 2 * P), jnp.bfloat16),
            pltpu.VMEM((NPAD,), jnp.float32),
            pltpu.VMEM((CHUNK, 16), jnp.float32),
            pltpu.VMEM((RPT // CHUNK, CHUNK), jnp.int32),
            pltpu.SemaphoreType.DMA,
            pltpu.SemaphoreType.DMA,
        ],
        compiler_params=pltpu.CompilerParams(needs_layout_passes=False, use_tc_tiling_on_sc=False),
    )


BN = 512


def _dense_body(aL_ref, aR_ref, x_ref, deg_ref, w1l0_ref, w1l1_ref,
                w1r_ref, b1_ref, w2_ref, b2_ref, s_ref):
    inv = 1.0 / jnp.maximum(deg_ref[...], 1.0)
    f32 = jnp.float32
    aL = aL_ref[...].astype(f32) * inv
    aR = aR_ref[...].astype(f32) * inv
    h = (jnp.dot(aL, w1l0_ref[...], preferred_element_type=f32)
         + jnp.dot(aR, w1l1_ref[...], preferred_element_type=f32)
         + jnp.dot(x_ref[...], w1r_ref[...], preferred_element_type=f32)
         + b1_ref[...])
    h = jnp.maximum(h, 0.0)
    s_ref[...] = (jnp.dot(h, w2_ref[...], preferred_element_type=f32)
                  + b2_ref[...])


def _make_dense_kernel():
    nb = NPAD // BN
    return pl.pallas_call(
        _dense_body,
        grid=(nb,),
        in_specs=[
            pl.BlockSpec((BN, 2 * P), lambda i: (i, 0)),
            pl.BlockSpec((BN, 2 * P), lambda i: (i, 0)),
            pl.BlockSpec((BN, D), lambda i: (i, 0)),
            pl.BlockSpec((BN, 1), lambda i: (i, 0)),
            pl.BlockSpec((2 * P, H), lambda i: (0, 0)),
            pl.BlockSpec((2 * P, H), lambda i: (0, 0)),
            pl.BlockSpec((D, H), lambda i: (0, 0)),
            pl.BlockSpec((1, H), lambda i: (0, 0)),
            pl.BlockSpec((H, 2), lambda i: (0, 0)),
            pl.BlockSpec((1, 2), lambda i: (0, 0)),
        ],
        out_specs=pl.BlockSpec((BN, 2), lambda i: (i, 0)),
        out_shape=jax.ShapeDtypeStruct((NPAD, 2), jnp.float32),
    )


def _seg2_body(s_hbm, src_hbm, dst_hbm, deg_hbm, iota_hbm,
               out_hbm,
               t_sh, s_v, src_v, dst_v, deg_v, t_l, rep_v, tb_v, out_v,
               iota_v):
    cid = lax.axis_index("c")
    sid = lax.axis_index("s")

    pltpu.sync_copy(s_hbm, s_v)
    pltpu.sync_copy(src_hbm.at[sid], src_v)
    pltpu.sync_copy(dst_hbm.at[sid], dst_v)
    pltpu.sync_copy(deg_hbm, deg_v)
    pltpu.sync_copy(iota_hbm, iota_v)

    zero16 = jnp.zeros((16,), jnp.float32)

    def zt(i, _):
        t_l[pl.ds(i * 16, 16)] = zero16
        return 0
    lax.fori_loop(0, RPT, zt, 0)

    def zrep(i, _):
        rep_v[i] = zero16
        return 0
    lax.fori_loop(0, CHUNK, zrep, 0)

    @pl.when(sid == 0)
    def _():
        for j in range(RPT // CHUNK):
            pltpu.sync_copy(rep_v, t_sh.at[pl.ds(j * CHUNK, CHUNK)])

    plsc.subcore_barrier()

    def body(c, _):
        for j in range(CHUNK // 16):
            s16 = src_v[c, pl.ds(j * 16, 16)]
            d16 = dst_v[c, pl.ds(j * 16, 16)]
            v = plsc.load_gather(s_v, [s16])
            plsc.addupdate_scatter(t_l, [d16], v)
        return 0
    lax.fori_loop(0, CPT, body, 0)

    for j in range(RPT // CHUNK):
        def rbody(i, _):
            rep_v[i] = t_l[pl.ds(j * CHUNK * 16 + i * 16, 16)]
            return 0
        lax.fori_loop(0, CHUNK, rbody, 0)
        pltpu.sync_copy(rep_v, t_sh.at[iota_v.at[j]], add=True)

    plsc.subcore_barrier()

    @pl.when(cid == 0)
    def _():
        pltpu.sync_copy(t_sh.at[pl.ds(sid * (RPT // 16), RPT // 16)], tb_v)
        base = sid * RPT
        i16 = lax.iota(jnp.int32, 16)

        def fbody(j, _):
            tt = tb_v[j]
            dd = deg_v[pl.ds(base + j * 16, 16)]
            s1 = plsc.load_gather(
                s_v, [lax.shift_left(i16 + (base + j * 16), 1) + 1])
            out_v[pl.ds(j * 16, 16)] = tt / jnp.maximum(dd, 1.0) + s1
            return 0
        lax.fori_loop(0, RPT // 16, fbody, 0)
        pltpu.sync_copy(out_v, out_hbm.at[pl.ds(base, RPT)])


def _make_seg2_kernel():
    return pl.kernel(
        _seg2_body,
        out_type=jax.ShapeDtypeStruct((NPAD,), jnp.float32),
        mesh=_sc_mesh(),
        scratch_types=[
            pltpu.VMEM_SHARED((NPAD // 16, 16), jnp.float32),
            pltpu.VMEM((NPAD * 2,), jnp.float32),
            pltpu.VMEM((CPT, CHUNK), jnp.int32),
            pltpu.VMEM((CPT, CHUNK), jnp.int32),
            pltpu.VMEM((NPAD,), jnp.float32),
            pltpu.VMEM((NPAD,), jnp.float32),
            pltpu.VMEM((CHUNK, 16), jnp.float32),
            pltpu.VMEM((RPT // 16, 16), jnp.float32),
            pltpu.VMEM((RPT,), jnp.float32),
            pltpu.VMEM((RPT // CHUNK, CHUNK), jnp.int32),
        ],
        compiler_params=pltpu.CompilerParams(needs_layout_passes=False, use_tc_tiling_on_sc=False),
    )


@jax.jit
def kernel(x, edge_index, W1_l, W1_r, b1, W2_l, W2_r, b2):
    npad = EP - E
    pad_src = (jnp.arange(npad, dtype=jnp.int32) * 37) % N
    pad_dst = N + (jnp.arange(npad, dtype=jnp.int32) % (NPAD - N))
    src = jnp.concatenate([edge_index[0].astype(jnp.int32), pad_src]) * 2
    dst = jnp.concatenate([edge_index[1].astype(jnp.int32), pad_dst])
    src3 = src.reshape(NS, CPT, CHUNK)
    dst3 = dst.reshape(NS, CPT, CHUNK)

    xf = x.astype(jnp.bfloat16).reshape(2 * N, 2 * P)
    zrow = jnp.zeros((RPT, 2 * P), jnp.bfloat16)
    iota = jnp.arange(NPAD // 16, dtype=jnp.int32).reshape(RPT // CHUNK, CHUNK)

    srcA = src.reshape(NS, CPTA, CA)
    dstA = dst.reshape(NS, CPTA, CA)
    aL, aR, deg2 = _make_aggr_kernel()(xf, srcA, dstA, zrow, iota)
    deg = deg2.reshape(NPAD, 1)

    w2 = jnp.concatenate([W2_l, W2_r], axis=0).T
    b2v = jnp.stack([jnp.zeros((), jnp.float32), b2[0]]).reshape(1, 2)
    w1lT = W1_l.T
    s = _make_dense_kernel()(
        aL, aR, x, deg, w1lT[:2 * P], w1lT[2 * P:], W1_r.T,
        b1.reshape(1, H), w2, b2v)

    outf = _make_seg2_kernel()(s.reshape(NPAD * 2), src3, dst3,
                               deg.reshape(NPAD), iota)
    return outf[:N].reshape(N, 1)
